# Initial kernel scaffold; baseline (speedup 1.0000x reference)
import jax
import jax.numpy as jnp
from jax.experimental import pallas as pl


def kernel(point, fc1_w, fc1_b, fc2_w, fc2_b):
    raise NotImplementedError("write your pallas kernel here")


# speedup vs baseline: 11.2730x; 11.2730x over previous
import functools

import jax
import jax.numpy as jnp
from jax import lax
from jax.experimental import pallas as pl
from jax.experimental.pallas import tpu as pltpu
from jax.experimental.pallas import tpu_sc as plsc

_K = 16
_RQ = 256
_RM = 512

_NC = 2
_NS = 16
_NW = _NC * _NS
_LANES = 16


def _topk_body(q_ref, keys_ref, idx_ref):
    b = pl.program_id(0)
    n = keys_ref.shape[2]
    q = q_ref[0]
    kx = keys_ref[0, 0:1, :]
    ky = keys_ref[0, 1:2, :]
    kz = keys_ref[0, 2:3, :]
    dx = q[:, 0:1] - kx
    dy = q[:, 1:2] - ky
    dz = q[:, 2:3] - kz
    d = (dx * dx + dy * dy) + dz * dz
    iota = lax.broadcasted_iota(jnp.int32, d.shape, 1)
    inf = jnp.float32(jnp.inf)
    big_i = jnp.int32(2**31 - 1)
    m_prev = jnp.full((q.shape[0], 1), -jnp.inf, jnp.float32)
    a_prev = jnp.full((q.shape[0], 1), -1, jnp.int32)
    cols = []
    for _ in range(_K):
        elig = (d > m_prev) | ((d == m_prev) & (iota > a_prev))
        dd = jnp.where(elig, d, inf)
        m = jnp.min(dd, axis=1, keepdims=True)
        ii = jnp.where(dd == m, iota, big_i)
        a = jnp.min(ii, axis=1, keepdims=True)
        cols.append(a)
        m_prev, a_prev = m, a
    idx_ref[0] = jnp.concatenate(cols, axis=1) + b * n


def _topk_indices(point, pt_t):
    b, n, _ = point.shape
    grid = (b, n // _RQ)
    return pl.pallas_call(
        _topk_body,
        grid=grid,
        in_specs=[
            pl.BlockSpec((1, _RQ, 3), lambda bi, i: (bi, i, 0)),
            pl.BlockSpec((1, 3, n), lambda bi, i: (bi, 0, 0)),
        ],
        out_specs=pl.BlockSpec((1, _RQ, _K), lambda bi, i: (bi, i, 0)),
        out_shape=jax.ShapeDtypeStruct((b, n, _K), jnp.int32),
    )(point, pt_t)


def _sc_gather(tab, idx2d):
    rows = idx2d.shape[0]
    rows_per_w = rows // _NW
    chunk = 16
    n_chunks = rows_per_w // chunk
    mesh = plsc.VectorSubcoreMesh(
        core_axis_name="c", subcore_axis_name="s",
        num_cores=_NC, num_subcores=_NS)

    @functools.partial(
        pl.kernel,
        out_type=jax.ShapeDtypeStruct((rows, 128, _LANES), jnp.float32),
        mesh=mesh,
        scratch_types=[
            pltpu.VMEM((chunk, 128), jnp.int32),
            pltpu.VMEM((chunk, 128, _LANES), jnp.float32),
            pltpu.SemaphoreType.DMA,
        ],
        compiler_params=pltpu.CompilerParams(use_tc_tiling_on_sc=False),
    )
    def k(tab_hbm, idx_hbm, out_hbm, idx_v, rows_v, sem):
        wid = lax.axis_index("s") * _NC + lax.axis_index("c")
        for c in range(n_chunks):
            base = wid * rows_per_w + c * chunk
            pltpu.sync_copy(idx_hbm.at[pl.ds(base, chunk)], idx_v)
            copies = [
                pltpu.async_copy(tab_hbm.at[idx_v.at[j]], rows_v.at[j], sem)
                for j in range(chunk)
            ]
            for cp in copies:
                cp.wait()
            pltpu.sync_copy(rows_v, out_hbm.at[pl.ds(base, chunk)])

    return k(tab, idx2d)


def _mlp_body(q_ref, g_ref, wa_ref, wc_ref, b1_ref, f2_ref, b2_ref, out_ref):
    q = q_ref[0]
    wa = wa_ref[...]
    wc = wc_ref[...]
    b1 = b1_ref[...]
    f2 = f2_ref[...]
    g = g_ref[0]

    def mat3(v0, v1, v2, w):
        return (v0 * w[0:1, :] + v1 * w[1:2, :]) + v2 * w[2:3, :]

    c = mat3(q[:, 0:1], q[:, 1:2], q[:, 2:3], wc) + b1
    acc = None
    for k in range(_K):
        o = _LANES * k
        h = mat3(g[:, o:o + 1], g[:, o + 1:o + 2], g[:, o + 2:o + 3], wa) + c
        s = h * (1.0 / (1.0 + jnp.exp(-h)))
        out = jnp.dot(s, f2, preferred_element_type=jnp.float32)
        acc = out if acc is None else jnp.maximum(acc, out)
    out_ref[0] = acc + b2_ref[...]


def _mlp(point, gathered, wa, wc, b1, f2, b2):
    b, n, _ = point.shape
    enc = f2.shape[1]
    hid = f2.shape[0]
    grid = (b, n // _RM)
    return pl.pallas_call(
        _mlp_body,
        grid=grid,
        in_specs=[
            pl.BlockSpec((1, _RM, 3), lambda bi, i: (bi, i, 0)),
            pl.BlockSpec((1, _RM, _K * _LANES), lambda bi, i: (bi, i, 0)),
            pl.BlockSpec((3, hid), lambda bi, i: (0, 0)),
            pl.BlockSpec((3, hid), lambda bi, i: (0, 0)),
            pl.BlockSpec((1, hid), lambda bi, i: (0, 0)),
            pl.BlockSpec((hid, enc), lambda bi, i: (0, 0)),
            pl.BlockSpec((1, enc), lambda bi, i: (0, 0)),
        ],
        out_specs=pl.BlockSpec((1, _RM, enc), lambda bi, i: (bi, i, 0)),
        out_shape=jax.ShapeDtypeStruct((b, n, enc), jnp.float32),
    )(point, gathered, wa, wc, b1, f2, b2)


def kernel(point, fc1_w, fc1_b, fc2_w, fc2_b):
    b, n, _ = point.shape
    hid = fc1_w.shape[0]
    enc = fc2_w.shape[0]

    pt_t = jnp.transpose(point, (0, 2, 1))
    idx = _topk_indices(point, pt_t)

    flat = point.reshape(b * n, 3)
    tab = jnp.pad(flat, ((0, 0), (0, _LANES - 3)))
    idx2d = idx.reshape(-1, 128)
    g = _sc_gather(tab, idx2d)
    gathered = g.reshape(b, n, _K * _LANES)

    wa = jnp.transpose(fc1_w[:, :3])
    wc = jnp.transpose(fc1_w[:, 3:] - fc1_w[:, :3])
    b1 = fc1_b.reshape(1, hid)
    f2 = jnp.transpose(fc2_w)
    b2 = fc2_b.reshape(1, enc)
    return _mlp(point, gathered, wa, wc, b1, f2, b2)


# speedup vs baseline: 20.8866x; 1.8528x over previous
import functools

import jax
import jax.numpy as jnp
from jax import lax
from jax.experimental import pallas as pl
from jax.experimental.pallas import tpu as pltpu
from jax.experimental.pallas import tpu_sc as plsc

_K = 16
_RQ = 256
_RM = 512

_NC = 2
_NS = 16
_NW = _NC * _NS
_LANES = 16


_DEPTH = 4
_SLOTS = 256


def _topk_body(q_ref, keys_ref, idx_ref, v_refs, i_refs):
    b = pl.program_id(0)
    n = keys_ref.shape[2]
    nchunks = n // _SLOTS
    rq = q_ref.shape[1]
    ngroups = rq // 8
    inf = jnp.float32(jnp.inf)
    big_i = jnp.int32(2**31 - 1)
    lane_g = lax.broadcasted_iota(jnp.int32, (8, _SLOTS), 1)

    def build_group(g, carry):
        rs = pl.ds(g * 8, 8)
        qg = q_ref[0, rs, :]
        qx, qy, qz = qg[:, 0:1], qg[:, 1:2], qg[:, 2:3]
        vals = [jnp.full((8, _SLOTS), inf) for _ in range(_DEPTH)]
        idxs = [jnp.full((8, _SLOTS), big_i, jnp.int32)
                for _ in range(_DEPTH)]
        for c in range(nchunks):
            sl = slice(c * _SLOTS, (c + 1) * _SLOTS)
            dx = qx - keys_ref[0, 0:1, sl]
            dy = qy - keys_ref[0, 1:2, sl]
            dz = qz - keys_ref[0, 2:3, sl]
            cur_v = (dx * dx + dy * dy) + dz * dz
            cur_i = lane_g + c * _SLOTS
            for lvl in range(_DEPTH):
                lt = (cur_v < vals[lvl]) | (
                    (cur_v == vals[lvl]) & (cur_i < idxs[lvl]))
                nv = jnp.where(lt, cur_v, vals[lvl])
                ni = jnp.where(lt, cur_i, idxs[lvl])
                cur_v = jnp.where(lt, vals[lvl], cur_v)
                cur_i = jnp.where(lt, idxs[lvl], cur_i)
                vals[lvl] = nv
                idxs[lvl] = ni
        for lvl in range(_DEPTH):
            v_refs[lvl][rs, :] = vals[lvl]
            i_refs[lvl][rs, :] = idxs[lvl]
        return carry

    lax.fori_loop(0, ngroups, build_group, 0)

    bad = jnp.zeros((rq, _SLOTS), jnp.bool_)
    cols = []
    for k in range(_K):
        v1 = v_refs[0][...]
        i1 = i_refs[0][...]
        if k >= _DEPTH:
            bad = bad | (v1 == inf)
        m = jnp.min(v1, axis=1, keepdims=True)
        ii = jnp.where(v1 == m, i1, big_i)
        a = jnp.min(ii, axis=1, keepdims=True)
        sel = i1 == a
        for t in range(_DEPTH - 1):
            v_refs[t][...] = jnp.where(sel, v_refs[t + 1][...], v_refs[t][...])
            i_refs[t][...] = jnp.where(sel, i_refs[t + 1][...], i_refs[t][...])
        v_refs[_DEPTH - 1][...] = jnp.where(sel, inf, v_refs[_DEPTH - 1][...])
        i_refs[_DEPTH - 1][...] = jnp.where(sel, big_i, i_refs[_DEPTH - 1][...])
        cols.append(a)
    idx_ref[0] = jnp.concatenate(cols, axis=1) + b * n

    @pl.when(jnp.any(bad))
    def _():
        q = q_ref[0]
        dx = q[:, 0:1] - keys_ref[0, 0:1, :]
        dy = q[:, 1:2] - keys_ref[0, 1:2, :]
        dz = q[:, 2:3] - keys_ref[0, 2:3, :]
        d = (dx * dx + dy * dy) + dz * dz
        iota = lax.broadcasted_iota(jnp.int32, d.shape, 1)
        m_prev = jnp.full((rq, 1), -jnp.inf)
        a_prev = jnp.full((rq, 1), -1, jnp.int32)
        cols2 = []
        for _ in range(_K):
            elig = (d > m_prev) | ((d == m_prev) & (iota > a_prev))
            dd = jnp.where(elig, d, inf)
            m = jnp.min(dd, axis=1, keepdims=True)
            ii = jnp.where(dd == m, iota, big_i)
            a = jnp.min(ii, axis=1, keepdims=True)
            cols2.append(a)
            m_prev, a_prev = m, a
        idx_ref[0] = jnp.concatenate(cols2, axis=1) + b * n


def _topk_indices(point, pt_t):
    b, n, _ = point.shape
    grid = (b, n // _RQ)
    return pl.pallas_call(
        _topk_body,
        grid=grid,
        in_specs=[
            pl.BlockSpec((1, _RQ, 3), lambda bi, i: (bi, i, 0)),
            pl.BlockSpec((1, 3, n), lambda bi, i: (bi, 0, 0)),
        ],
        out_specs=pl.BlockSpec((1, _RQ, _K), lambda bi, i: (bi, i, 0)),
        out_shape=jax.ShapeDtypeStruct((b, n, _K), jnp.int32),
        scratch_shapes=[
            [pltpu.VMEM((_RQ, _SLOTS), jnp.float32) for _ in range(_DEPTH)],
            [pltpu.VMEM((_RQ, _SLOTS), jnp.int32) for _ in range(_DEPTH)],
        ],
    )(point, pt_t)


def _sc_gather(tab, idx2d):
    rows = idx2d.shape[0]
    rows_per_w = rows // _NW
    chunk = 16
    n_chunks = rows_per_w // chunk
    mesh = plsc.VectorSubcoreMesh(
        core_axis_name="c", subcore_axis_name="s",
        num_cores=_NC, num_subcores=_NS)

    @functools.partial(
        pl.kernel,
        out_type=jax.ShapeDtypeStruct((rows, 128, _LANES), jnp.float32),
        mesh=mesh,
        scratch_types=[
            pltpu.VMEM((chunk, 128), jnp.int32),
            pltpu.VMEM((chunk, 128, _LANES), jnp.float32),
            pltpu.SemaphoreType.DMA,
        ],
        compiler_params=pltpu.CompilerParams(use_tc_tiling_on_sc=False),
    )
    def k(tab_hbm, idx_hbm, out_hbm, idx_v, rows_v, sem):
        wid = lax.axis_index("s") * _NC + lax.axis_index("c")
        for c in range(n_chunks):
            base = wid * rows_per_w + c * chunk
            pltpu.sync_copy(idx_hbm.at[pl.ds(base, chunk)], idx_v)
            copies = [
                pltpu.async_copy(tab_hbm.at[idx_v.at[j]], rows_v.at[j], sem)
                for j in range(chunk)
            ]
            for cp in copies:
                cp.wait()
            pltpu.sync_copy(rows_v, out_hbm.at[pl.ds(base, chunk)])

    return k(tab, idx2d)


def _mlp_body(q_ref, g_ref, wa_ref, wc_ref, b1_ref, f2_ref, b2_ref, out_ref):
    q = q_ref[0]
    wa = wa_ref[...]
    wc = wc_ref[...]
    b1 = b1_ref[...]
    f2 = f2_ref[...]
    g = g_ref[0]

    def mat3(v0, v1, v2, w):
        return (v0 * w[0:1, :] + v1 * w[1:2, :]) + v2 * w[2:3, :]

    c = mat3(q[:, 0:1], q[:, 1:2], q[:, 2:3], wc) + b1
    acc = None
    for k in range(_K):
        o = _LANES * k
        h = mat3(g[:, o:o + 1], g[:, o + 1:o + 2], g[:, o + 2:o + 3], wa) + c
        s = h * (1.0 / (1.0 + jnp.exp(-h)))
        out = jnp.dot(s, f2, preferred_element_type=jnp.float32)
        acc = out if acc is None else jnp.maximum(acc, out)
    out_ref[0] = acc + b2_ref[...]


def _mlp(point, gathered, wa, wc, b1, f2, b2):
    b, n, _ = point.shape
    enc = f2.shape[1]
    hid = f2.shape[0]
    grid = (b, n // _RM)
    return pl.pallas_call(
        _mlp_body,
        grid=grid,
        in_specs=[
            pl.BlockSpec((1, _RM, 3), lambda bi, i: (bi, i, 0)),
            pl.BlockSpec((1, _RM, _K * _LANES), lambda bi, i: (bi, i, 0)),
            pl.BlockSpec((3, hid), lambda bi, i: (0, 0)),
            pl.BlockSpec((3, hid), lambda bi, i: (0, 0)),
            pl.BlockSpec((1, hid), lambda bi, i: (0, 0)),
            pl.BlockSpec((hid, enc), lambda bi, i: (0, 0)),
            pl.BlockSpec((1, enc), lambda bi, i: (0, 0)),
        ],
        out_specs=pl.BlockSpec((1, _RM, enc), lambda bi, i: (bi, i, 0)),
        out_shape=jax.ShapeDtypeStruct((b, n, enc), jnp.float32),
    )(point, gathered, wa, wc, b1, f2, b2)


def kernel(point, fc1_w, fc1_b, fc2_w, fc2_b):
    b, n, _ = point.shape
    hid = fc1_w.shape[0]
    enc = fc2_w.shape[0]

    pt_t = jnp.transpose(point, (0, 2, 1))
    idx = _topk_indices(point, pt_t)

    flat = point.reshape(b * n, 3)
    tab = jnp.pad(flat, ((0, 0), (0, _LANES - 3)))
    idx2d = idx.reshape(-1, 128)
    g = _sc_gather(tab, idx2d)
    gathered = g.reshape(b, n, _K * _LANES)

    wa = jnp.transpose(fc1_w[:, :3])
    wc = jnp.transpose(fc1_w[:, 3:] - fc1_w[:, :3])
    b1 = fc1_b.reshape(1, hid)
    f2 = jnp.transpose(fc2_w)
    b2 = fc2_b.reshape(1, enc)
    return _mlp(point, gathered, wa, wc, b1, f2, b2)
